# Initial kernel scaffold; baseline (speedup 1.0000x reference)
import jax
import jax.numpy as jnp
from jax.experimental import pallas as pl


def kernel(x, wf0, cb0, wf1, cb1, wf2, cb2, wf3, cb3, fw1, fb1, fw2, fb2, fw3, fb3):
    raise NotImplementedError("write your pallas kernel here")


# speedup vs baseline: 6.2159x; 6.2159x over previous
import numpy as np
import jax
import jax.numpy as jnp
from jax.experimental import pallas as pl
from jax.experimental.pallas import tpu as pltpu

NB = 16
B0 = 128
TAIL = 128

HA, WPA, RA, SA = 28, 30, 840, 896
HB, WPB, RB, SB = 14, 16, 224, 256
LA, LB = NB * SA, NB * SB
TOTA, TOTB = B0 + LA + TAIL, B0 + LB + TAIL
LANE = 128
PO = 7


def _interior_mask(s, h, w):
    wp = w + 2
    m = np.zeros((1, NB * s), np.float32)
    for i in range(NB):
        for y in range(h):
            m[0, i * s + y * wp: i * s + y * wp + w] = 1.0
    return jnp.asarray(m)


def _pool_sel(h, wp, oh, ow, ostride, ocols, scale):
    s = np.zeros((h * wp, ocols), np.float32)
    for yo in range(oh):
        for xo in range(ow):
            s[2 * yo * wp + 2 * xo, yo * ostride + xo] = scale
    return jnp.asarray(s, jnp.bfloat16)


def _feat_body(x_ref, wf0, cb0, wf1, cb1, wf2, cb2, wf3, cb3,
               mA, mB, selmax, selavg, o_ref,
               a0, a1, a2, a3, a4, col0, col1, col2, col3, mstk, astk):

    def conv(src_read, col, w, b, msk, dst, cin, wp, length):
        for k in range(9):
            off = -wp - 1 + (k // 3) * wp + (k % 3)
            col[pl.ds(k * cin, cin), :] = src_read(off, length)
        acc = jnp.dot(w[...], col[...], preferred_element_type=jnp.float32)
        acc = jnp.maximum(acc + b[...], 0.0) * msk[...]
        cout = acc.shape[0]
        dst[:, pl.ds(0, B0)] = jnp.zeros((cout, B0), jnp.bfloat16)
        dst[:, pl.ds(B0, length)] = acc.astype(jnp.bfloat16)
        dst[:, pl.ds(B0 + length, TAIL)] = jnp.zeros((cout, TAIL), jnp.bfloat16)

    xr = lambda off, ln: x_ref[0, :, pl.ds(B0 + off, ln)]
    conv(xr, col0, wf0, cb0, mA, a0, 3, WPA, LA)
    a0r = lambda off, ln: a0[:, pl.ds(B0 + off, ln)]
    conv(a0r, col1, wf1, cb1, mA, a1, 8, WPA, LA)

    m = jnp.maximum(
        jnp.maximum(a1[:, pl.ds(B0, LA)], a1[:, pl.ds(B0 + 1, LA)]),
        jnp.maximum(a1[:, pl.ds(B0 + WPA, LA)], a1[:, pl.ds(B0 + WPA + 1, LA)]))
    for i in range(NB):
        mstk[pl.ds(i * 8, 8), :] = m[:, i * SA:i * SA + RA]
    pooled = jnp.dot(mstk[...], selmax[...], preferred_element_type=jnp.float32)
    a2[...] = jnp.zeros((8, TOTB), jnp.bfloat16)
    for i in range(NB):
        a2[:, pl.ds(B0 + i * SB, RB)] = pooled[i * 8:(i + 1) * 8, :].astype(jnp.bfloat16)

    a2r = lambda off, ln: a2[:, pl.ds(B0 + off, ln)]
    conv(a2r, col2, wf2, cb2, mB, a3, 8, WPB, LB)
    a3r = lambda off, ln: a3[:, pl.ds(B0 + off, ln)]
    conv(a3r, col3, wf3, cb3, mB, a4, 16, WPB, LB)

    s4 = (a4[:, pl.ds(B0, LB)].astype(jnp.float32)
          + a4[:, pl.ds(B0 + 1, LB)].astype(jnp.float32)
          + a4[:, pl.ds(B0 + WPB, LB)].astype(jnp.float32)
          + a4[:, pl.ds(B0 + WPB + 1, LB)].astype(jnp.float32))
    for i in range(NB):
        astk[pl.ds(i * 16, 16), :] = s4[:, i * SB:i * SB + RB].astype(jnp.bfloat16)
    res = jnp.dot(astk[...], selavg[...], preferred_element_type=jnp.float32)
    for i in range(NB):
        o_ref[0, i] = res[i * 16:(i + 1) * 16, :]


def _cls_body(f_ref, w1, b1, w2, b2, w3, b3, o_ref):
    h = jnp.maximum(jnp.dot(f_ref[...], w1[...],
                            preferred_element_type=jnp.float32) + b1[...], 0.0)
    h = jnp.maximum(jnp.dot(h, w2[...],
                            preferred_element_type=jnp.float32) + b2[...], 0.0)
    o_ref[...] = jnp.dot(h, w3[...], preferred_element_type=jnp.float32) + b3[...]


def kernel(x, wf0, cb0, wf1, cb1, wf2, cb2, wf3, cb3,
           fw1, fb1, fw2, fb2, fw3, fb3):
    n = x.shape[0]
    nb2 = n + (-n % NB)
    xp = jnp.pad(x.astype(jnp.float32), ((0, nb2 - n), (0, 0), (0, 0), (0, 2)))
    xf = xp.reshape(nb2, 3, RA)
    xf = jnp.pad(xf, ((0, 0), (0, 0), (0, SA - RA)))
    xf = xf.reshape(nb2 // NB, NB, 3, SA).transpose(0, 2, 1, 3)
    xf = xf.reshape(nb2 // NB, 3, LA)
    xf = jnp.pad(xf, ((0, 0), (0, 0), (B0, TAIL))).astype(jnp.bfloat16)

    mA = _interior_mask(SA, HA, HA)
    mB = _interior_mask(SB, HB, HB)
    selmax = _pool_sel(HA, WPA, HB, HB, WPB, RB, 1.0)
    selavg = _pool_sel(HB, WPB, PO, PO, PO, LANE, 0.25)

    grid = nb2 // NB
    args = (xf, wf0, cb0, wf1, cb1, wf2, cb2, wf3, cb3, mA, mB, selmax, selavg)
    in_specs = [pl.BlockSpec((1, 3, TOTA), lambda b: (b, 0, 0))]
    in_specs += [pl.BlockSpec(a.shape, lambda b: (0, 0)) for a in args[1:]]
    feats = pl.pallas_call(
        _feat_body,
        out_shape=jax.ShapeDtypeStruct((grid, NB, 16, LANE), jnp.float32),
        grid=(grid,),
        in_specs=in_specs,
        out_specs=pl.BlockSpec((1, NB, 16, LANE), lambda b: (b, 0, 0, 0)),
        scratch_shapes=[
            pltpu.VMEM((8, TOTA), jnp.bfloat16),
            pltpu.VMEM((8, TOTA), jnp.bfloat16),
            pltpu.VMEM((8, TOTB), jnp.bfloat16),
            pltpu.VMEM((16, TOTB), jnp.bfloat16),
            pltpu.VMEM((16, TOTB), jnp.bfloat16),
            pltpu.VMEM((27, LA), jnp.bfloat16),
            pltpu.VMEM((72, LA), jnp.bfloat16),
            pltpu.VMEM((72, LB), jnp.bfloat16),
            pltpu.VMEM((144, LB), jnp.bfloat16),
            pltpu.VMEM((NB * 8, RA), jnp.bfloat16),
            pltpu.VMEM((NB * 16, RB), jnp.bfloat16),
        ],
        compiler_params=pltpu.CompilerParams(dimension_semantics=("parallel",)),
    )(*args)

    kk = PO * PO
    w1p = jnp.pad(fw1.reshape(16, kk, 64), ((0, 0), (0, LANE - kk), (0, 0)))
    w1p = w1p.reshape(16 * LANE, 64)
    w3p = jnp.pad(fw3, ((0, 0), (0, LANE - fw3.shape[1])))
    b3p = jnp.pad(fb3, ((0, 0), (0, LANE - fb3.shape[1])))
    f = feats.reshape(nb2, 16 * LANE)
    nb3 = nb2 + (-nb2 % 128)
    f = jnp.pad(f, ((0, nb3 - nb2), (0, 0)))
    cargs = (f, w1p, fb1, fw2, fb2, w3p, b3p)
    cspecs = [pl.BlockSpec((128, 16 * LANE), lambda b: (b, 0))]
    cspecs += [pl.BlockSpec(a.shape, lambda b: (0, 0)) for a in cargs[1:]]
    out = pl.pallas_call(
        _cls_body,
        out_shape=jax.ShapeDtypeStruct((nb3, LANE), jnp.float32),
        grid=(nb3 // 128,),
        in_specs=cspecs,
        out_specs=pl.BlockSpec((128, LANE), lambda b: (b, 0)),
        compiler_params=pltpu.CompilerParams(dimension_semantics=("parallel",)),
    )(*cargs)
    return out[:n, :10]
